# Initial kernel scaffold; baseline (speedup 1.0000x reference)
import jax
import jax.numpy as jnp
from jax.experimental import pallas as pl


def kernel(codes, code_embed_weight):
    raise NotImplementedError("write your pallas kernel here")


# speedup vs baseline: 1.2775x; 1.2775x over previous
import functools

import jax
import jax.numpy as jnp
from jax import lax
from jax.experimental import pallas as pl
from jax.experimental.pallas import tpu as pltpu
from jax.experimental.pallas import tpu_sc as plsc

EMBED_D = 128
CHUNK = 128
NUM_CORES = 2
NUM_SUBCORES = 16
NW = NUM_CORES * NUM_SUBCORES


def _make_gather(n_rows: int, n_chunks: int):
  mesh = plsc.VectorSubcoreMesh(core_axis_name="c", subcore_axis_name="s")

  @functools.partial(
      pl.kernel,
      out_type=jax.ShapeDtypeStruct((n_rows, EMBED_D), jnp.float32),
      mesh=mesh,
      scratch_types=[
          pltpu.VMEM((n_chunks, CHUNK), jnp.int32),
          pltpu.VMEM((2, CHUNK, EMBED_D), jnp.float32),
          pltpu.SemaphoreType.DMA,
          pltpu.SemaphoreType.DMA,
      ],
  )
  def gather_kernel(table_hbm, idx_hbm, out_hbm, idx_v, rows_v, sem0, sem1):
    wid = lax.axis_index("s") * NUM_CORES + lax.axis_index("c")
    row_base = wid * (n_chunks * CHUNK)
    pltpu.sync_copy(idx_hbm.at[wid], idx_v)
    sems = (sem0, sem1)

    def start(j, b):
      pltpu.async_copy(table_hbm.at[idx_v.at[j]], rows_v.at[b], sems[b])

    def wait(b):
      pltpu.make_async_copy(
          table_hbm.at[idx_v.at[0]], rows_v.at[b], sems[b]).wait()

    start(0, 0)
    start(1, 1)

    def outer(g, _):
      for b in range(2):
        j = g * 2 + b
        wait(b)
        nxt = j + 2
        pltpu.sync_copy(rows_v.at[b],
                        out_hbm.at[pl.ds(row_base + j * CHUNK, CHUNK)])

        @pl.when(nxt < n_chunks)
        def _():
          start(nxt, b)

      return _

    lax.fori_loop(0, n_chunks // 2, outer, None)

  return gather_kernel


def kernel(codes, code_embed_weight):
  bsz, seq = codes.shape
  d = code_embed_weight.shape[1]
  idx = codes.reshape(-1).astype(jnp.int32)
  n_rows = idx.shape[0]
  n_chunks = n_rows // (NW * CHUNK)
  idx3 = idx.reshape(NW, n_chunks, CHUNK)
  out = _make_gather(n_rows, n_chunks)(code_embed_weight, idx3)
  return out.reshape(bsz, seq, d)


# speedup vs baseline: 1.2836x; 1.0048x over previous
import functools

import jax
import jax.numpy as jnp
from jax import lax
from jax.experimental import pallas as pl
from jax.experimental.pallas import tpu as pltpu
from jax.experimental.pallas import tpu_sc as plsc

EMBED_D = 128
CHUNK = 128
NBUF = 5
AHEAD = 2
NUM_CORES = 2
NUM_SUBCORES = 16
NW = NUM_CORES * NUM_SUBCORES


def _make_gather(n_rows: int, n_chunks: int):
  mesh = plsc.VectorSubcoreMesh(core_axis_name="c", subcore_axis_name="s")

  @functools.partial(
      pl.kernel,
      out_type=jax.ShapeDtypeStruct((n_rows, EMBED_D), jnp.float32),
      mesh=mesh,
      scratch_types=[
          pltpu.VMEM((n_chunks, CHUNK), jnp.int32),
          pltpu.VMEM((NBUF, CHUNK, EMBED_D), jnp.float32),
      ]
      + [pltpu.SemaphoreType.DMA] * (2 * NBUF),
  )
  def gather_kernel(table_hbm, idx_hbm, out_hbm, idx_v, rows_v, *sems):
    g_sems = sems[:NBUF]
    o_sems = sems[NBUF:]
    wid = lax.axis_index("s") * NUM_CORES + lax.axis_index("c")
    row_base = wid * (n_chunks * CHUNK)
    pltpu.sync_copy(idx_hbm.at[wid], idx_v)

    def start_gather(j, b):
      pltpu.async_copy(table_hbm.at[idx_v.at[j]], rows_v.at[b], g_sems[b])

    def wait_gather(b):
      pltpu.make_async_copy(
          table_hbm.at[idx_v.at[0]], rows_v.at[b], g_sems[b]).wait()

    def start_out(j, b):
      pltpu.async_copy(rows_v.at[b],
                       out_hbm.at[pl.ds(row_base + j * CHUNK, CHUNK)],
                       o_sems[b])

    def wait_out(b):
      pltpu.make_async_copy(
          rows_v.at[b], out_hbm.at[pl.ds(row_base, CHUNK)], o_sems[b]).wait()

    for j in range(AHEAD):
      start_gather(j, j)

    def superstep(g, carry):
      for k in range(NBUF):
        j = g * NBUF + k
        b = k
        wait_gather(b)
        start_out(j, b)
        nxt = j + AHEAD
        bn = (k + AHEAD) % NBUF

        @pl.when(jnp.logical_and(nxt < n_chunks, nxt >= NBUF))
        def _():
          wait_out(bn)

        @pl.when(nxt < n_chunks)
        def _():
          start_gather(nxt, bn)

      return carry

    lax.fori_loop(0, n_chunks // NBUF, superstep, None)

    for b in range(NBUF):
      wait_out(b)

  return gather_kernel


def kernel(codes, code_embed_weight):
  bsz, seq = codes.shape
  d = code_embed_weight.shape[1]
  idx = codes.reshape(-1).astype(jnp.int32)
  n_rows = idx.shape[0]
  n_chunks = n_rows // (NW * CHUNK)
  idx3 = idx.reshape(NW, n_chunks, CHUNK)
  out = _make_gather(n_rows, n_chunks)(code_embed_weight, idx3)
  return out.reshape(bsz, seq, d)


# speedup vs baseline: 2.2915x; 1.7852x over previous
import functools

import jax
import jax.numpy as jnp
from jax import lax
from jax.experimental import pallas as pl
from jax.experimental.pallas import tpu as pltpu
from jax.experimental.pallas import tpu_sc as plsc

NUM_CORES = 2
NUM_SUBCORES = 16
NW = NUM_CORES * NUM_SUBCORES
SLAB = 8


def _make_gather(n_batch: int, seq: int, d: int):
  per_w = n_batch // NW
  n_slabs = per_w // SLAB
  mesh = plsc.VectorSubcoreMesh(core_axis_name="c", subcore_axis_name="s")

  @functools.partial(
      pl.kernel,
      out_type=jax.ShapeDtypeStruct((n_batch, seq, d), jnp.float32),
      mesh=mesh,
      scratch_types=[
          pltpu.VMEM((per_w, seq), jnp.int32),
          pltpu.VMEM((2, SLAB, seq, d), jnp.float32),
          pltpu.SemaphoreType.DMA,
          pltpu.SemaphoreType.DMA,
          pltpu.SemaphoreType.DMA,
          pltpu.SemaphoreType.DMA,
      ],
  )
  def gather_kernel(table_hbm, idx_hbm, out_hbm, idx_v, rows_v,
                    gs0, gs1, os0, os1):
    g_sems = (gs0, gs1)
    o_sems = (os0, os1)
    wid = lax.axis_index("s") * NUM_CORES + lax.axis_index("c")
    elem_base = wid * per_w
    pltpu.sync_copy(idx_hbm.at[wid], idx_v)

    def start_slab(si, b):
      for i in range(SLAB):
        pltpu.async_copy(table_hbm.at[idx_v.at[si * SLAB + i]],
                         rows_v.at[b].at[i], g_sems[b])

    def wait_slab(b):
      pltpu.make_async_copy(
          out_hbm.at[pl.ds(elem_base, SLAB)], rows_v.at[b], g_sems[b]).wait()

    def start_out(si, b):
      pltpu.async_copy(rows_v.at[b],
                       out_hbm.at[pl.ds(elem_base + si * SLAB, SLAB)],
                       o_sems[b])

    def wait_out(b):
      pltpu.make_async_copy(
          rows_v.at[b], out_hbm.at[pl.ds(elem_base, SLAB)], o_sems[b]).wait()

    start_slab(0, 0)

    def superstep(g, carry):
      for k in range(2):
        si = g * 2 + k
        b = k
        nxt = si + 1
        bn = (k + 1) % 2

        @pl.when(jnp.logical_and(nxt < n_slabs, si >= 1))
        def _():
          wait_out(bn)

        @pl.when(nxt < n_slabs)
        def _():
          start_slab(nxt, bn)

        wait_slab(b)
        start_out(si, b)
      return carry

    lax.fori_loop(0, n_slabs // 2, superstep, None)
    wait_out(0)
    wait_out(1)

  return gather_kernel


def kernel(codes, code_embed_weight):
  bsz, seq = codes.shape
  d = code_embed_weight.shape[1]
  idx3 = codes.astype(jnp.int32).reshape(NW, bsz // NW, seq)
  return _make_gather(bsz, seq, d)(code_embed_weight, idx3)


# speedup vs baseline: 4.0519x; 1.7682x over previous
import functools

import jax
import jax.numpy as jnp
from jax import lax
from jax.experimental import pallas as pl
from jax.experimental.pallas import tpu as pltpu
from jax.experimental.pallas import tpu_sc as plsc

EMBED_D = 128
CHUNK = 128
NBUF = 5
AHEAD = 2
NUM_CORES = 2
NUM_SUBCORES = 16
NW = NUM_CORES * NUM_SUBCORES


def _make_gather(n_rows: int, n_chunks: int):
  mesh = plsc.VectorSubcoreMesh(core_axis_name="c", subcore_axis_name="s")

  @functools.partial(
      pl.kernel,
      out_type=jax.ShapeDtypeStruct((n_rows, EMBED_D), jnp.float32),
      mesh=mesh,
      scratch_types=[
          pltpu.VMEM((n_chunks, CHUNK), jnp.int32),
          pltpu.VMEM((NBUF, CHUNK, EMBED_D), jnp.float32),
      ]
      + [pltpu.SemaphoreType.DMA] * (2 * NBUF),
  )
  def gather_kernel(table_hbm, idx_hbm, out_hbm, idx_v, rows_v, *sems):
    g_sems = sems[:NBUF]
    o_sems = sems[NBUF:]
    wid = lax.axis_index("s") * NUM_CORES + lax.axis_index("c")
    row_base = wid * (n_chunks * CHUNK)
    pltpu.sync_copy(idx_hbm.at[wid], idx_v)

    def start_gather(j, b):
      pltpu.async_copy(table_hbm.at[idx_v.at[j]], rows_v.at[b], g_sems[b])

    def wait_gather(b):
      pltpu.make_async_copy(
          table_hbm.at[idx_v.at[0]], rows_v.at[b], g_sems[b]).wait()

    def start_out(j, b):
      pltpu.async_copy(rows_v.at[b],
                       out_hbm.at[pl.ds(row_base + j * CHUNK, CHUNK)],
                       o_sems[b])

    def wait_out(b):
      pltpu.make_async_copy(
          rows_v.at[b], out_hbm.at[pl.ds(row_base, CHUNK)], o_sems[b]).wait()

    for j in range(AHEAD):
      start_gather(j, j)

    def superstep(g, carry):
      for k in range(NBUF):
        j = g * NBUF + k
        b = k
        wait_gather(b)
        start_out(j, b)
        nxt = j + AHEAD
        bn = (k + AHEAD) % NBUF

        @pl.when(jnp.logical_and(nxt < n_chunks, nxt >= NBUF))
        def _():
          wait_out(bn)

        @pl.when(nxt < n_chunks)
        def _():
          start_gather(nxt, bn)

      return carry

    lax.fori_loop(0, n_chunks // NBUF, superstep, None)

    for b in range(NBUF):
      wait_out(b)

  return gather_kernel


def kernel(codes, code_embed_weight):
  bsz, seq = codes.shape
  d = code_embed_weight.shape[1]
  idx = codes.T.astype(jnp.int32).reshape(-1)
  n_rows = idx.shape[0]
  n_chunks = n_rows // (NW * CHUNK)
  idx3 = idx.reshape(NW, n_chunks, CHUNK)
  out = _make_gather(n_rows, n_chunks)(code_embed_weight, idx3)
  return out.reshape(seq, bsz, d).transpose(1, 0, 2)
